# Initial kernel scaffold; baseline (speedup 1.0000x reference)
import jax
import jax.numpy as jnp
from jax.experimental import pallas as pl


def kernel(x, W, b):
    raise NotImplementedError("write your pallas kernel here")


# speedup vs baseline: 35.4344x; 35.4344x over previous
import functools
import numpy as np
import jax
import jax.numpy as jnp
from jax.experimental import pallas as pl

_ATOMS = (8, 16, 32, 64)
_OFFS = (0, 8, 24, 56)
_KS = (2, 4, 8, 16)
_OUT_CH = 120
_R = 8


def _conn_t_mats():
    mats = []
    for a in _ATOMS[:-1]:
        nxt = 2 * a
        m = np.zeros((nxt, a), np.float32)
        for j in range(a):
            for t in range(4):
                m[(2 * j + t) % nxt, j] = 1.0
        mats.append(jnp.asarray(m))
    return mats


def _body(xa_ref, xb_ref, w_ref, b_ref, c0_ref, c1_ref, c2_ref, out_ref,
          *, hw: int, n: int, nv: int, nrows_blk: int):
    xw = jnp.concatenate([xa_ref[0], xb_ref[0]], axis=1)
    acc = jnp.zeros((_OUT_CH, n), jnp.float32)
    for dy in range(3):
        for dx in range(3):
            off = dy * hw + dx
            xs = xw[:, off:off + n]
            wk = w_ref[dy * 3 + dx]
            acc = acc + jax.lax.dot_general(
                wk, xs, (((1,), (0,)), ((), ())),
                preferred_element_type=jnp.float32)
    acc = acc + b_ref[:]

    conn_refs = (None, c0_ref, c1_ref, c2_ref)
    prev_dm = None
    for li, (na, off, k) in enumerate(zip(_ATOMS, _OFFS, _KS)):
        a = acc[off:off + na, :]
        if prev_dm is not None:
            gp = jax.lax.dot_general(
                conn_refs[li][:], prev_dm, (((1,), (0,)), ((), ())),
                preferred_element_type=jnp.float32)
            a = a * (gp > 0).astype(jnp.float32)
        key = jnp.abs(a)
        iota = jax.lax.broadcasted_iota(jnp.int32, (na, n), 0)
        sel = jnp.zeros((na, n), jnp.bool_)
        for _ in range(k):
            m = jnp.max(key, axis=0, keepdims=True)
            cand = jnp.where(key == m, iota, na)
            idx = jnp.min(cand, axis=0, keepdims=True)
            chosen = iota == idx
            sel = jnp.logical_or(sel, chosen)
            key = jnp.where(chosen, -1.0, key)
        masked = jnp.where(sel, a, 0.0)
        prev_dm = (masked != 0).astype(jnp.float32)
        for r in range(nrows_blk):
            out_ref[0, off:off + na, r, :] = masked[:, r * hw:r * hw + nv]


def kernel(x, W, b):
    bsz, in_ch, h, hw = x.shape
    nv = hw - 2
    nrows = h - 2
    nch = pl.cdiv(nrows, _R)
    n = _R * hw
    halo = 2 * hw + 2
    tail = n // 2
    assert tail >= halo and _R * nch == h
    flat = h * hw
    max_tail_blk = flat // tail - 1

    x3 = x.reshape(bsz, in_ch, flat)
    w9 = jnp.transpose(W, (2, 3, 0, 1)).reshape(9, _OUT_CH, in_ch)
    b2 = b.reshape(_OUT_CH, 1)
    c0, c1, c2 = _conn_t_mats()

    body = functools.partial(_body, hw=hw, n=n, nv=nv, nrows_blk=_R)

    out = pl.pallas_call(
        body,
        grid=(bsz, nch),
        in_specs=[
            pl.BlockSpec((1, in_ch, n), lambda bi, c: (bi, 0, c)),
            pl.BlockSpec(
                (1, in_ch, tail),
                lambda bi, c: (bi, 0, jnp.minimum(2 * c + 2, max_tail_blk))),
            pl.BlockSpec((9, _OUT_CH, in_ch), lambda bi, c: (0, 0, 0)),
            pl.BlockSpec((_OUT_CH, 1), lambda bi, c: (0, 0)),
            pl.BlockSpec((16, 8), lambda bi, c: (0, 0)),
            pl.BlockSpec((32, 16), lambda bi, c: (0, 0)),
            pl.BlockSpec((64, 32), lambda bi, c: (0, 0)),
        ],
        out_specs=pl.BlockSpec((1, _OUT_CH, _R, nv),
                               lambda bi, c: (bi, 0, c, 0)),
        out_shape=jax.ShapeDtypeStruct((bsz, _OUT_CH, nrows, nv), jnp.float32),
    )(x3, x3, w9, b2, c0, c1, c2)
    return out
